# baseline (speedup 1.0000x reference)
import jax
import jax.numpy as jnp
from jax import lax
from jax.experimental import pallas as pl
from jax.experimental.pallas import tpu as pltpu
from jax.experimental.pallas import tpu_sc as plsc

_MAXSEQLEN = 2048
_B = 8
_H = 1024
_TOTAL = _B * _MAXSEQLEN // 2
_S = 2048
_RBLK = 256
_NBLK = _TOTAL // _RBLK
_NSC = _S // _RBLK
_NC = 2
_NS = 16
_NW = _NC * _NS
_ROWS_PER_W = _S // _NW
_CHUNK = 16
_NCHUNK = _ROWS_PER_W // _CHUNK
_NBUF = 4
_LANES = 16


def _sc_body(flat_hbm, cum_hbm, delta_hbm, out_hbm,
             tbl_v, idx_v, rows_v, *sems):
    wid = lax.axis_index("s") * _NC + lax.axis_index("c")
    base = pl.multiple_of(wid * _ROWS_PER_W, _ROWS_PER_W)

    pltpu.sync_copy(cum_hbm, tbl_v.at[0])
    pltpu.sync_copy(delta_hbm, tbl_v.at[1])

    for g in range(_ROWS_PER_W // _LANES):
        pos = base + g * _LANES + lax.iota(jnp.int32, _LANES)
        acc = pos
        for j in range(_B):
            cum_j = tbl_v[0, j, :]
            dlt_j = tbl_v[1, j, :]
            acc = acc + jnp.where(pos >= cum_j, dlt_j, 0)
        gpc = _CHUNK // _LANES
        idx_v[g // gpc, pl.ds((g % gpc) * _LANES, _LANES)] = acc

    gsems = sems[:_NBUF]
    ssems = sems[_NBUF:]

    def start_gather(c, buf):
        cp = pltpu.make_async_copy(
            flat_hbm.at[idx_v.at[c]], rows_v.at[buf], gsems[buf])
        cp.start()
        return cp

    g_handles = [None] * _NCHUNK
    s_handles = [None] * _NBUF
    for c in range(min(_NBUF - 1, _NCHUNK)):
        g_handles[c] = start_gather(c, c % _NBUF)
    for c in range(_NCHUNK):
        buf = c % _NBUF
        g_handles[c].wait()
        cp = pltpu.make_async_copy(
            rows_v.at[buf],
            out_hbm.at[pl.ds(base + c * _CHUNK, _CHUNK)],
            ssems[buf])
        cp.start()
        s_handles[buf] = cp
        nxt = c + _NBUF - 1
        if nxt < _NCHUNK:
            nb = nxt % _NBUF
            if s_handles[nb] is not None:
                s_handles[nb].wait()
                s_handles[nb] = None
            g_handles[nxt] = start_gather(nxt, nb)
    for buf in range(_NBUF):
        if s_handles[buf] is not None:
            s_handles[buf].wait()


def _tc_body(src_ref, hard_ref, cum_ref, dlt_ref,
             flat_any, sc_any, out_any, sem_bulk, sem_row):
    for i in range(_NBLK):
        dst = out_any.at[pl.ds(i * _RBLK, _RBLK)]
        if i < _NSC:
            pltpu.make_async_copy(
                sc_any.at[pl.ds(i * _RBLK, _RBLK)], dst, sem_bulk).start()
        else:
            @pl.when(hard_ref[i] == 0)
            def _(i=i, dst=dst):
                srow = pl.multiple_of(src_ref[i] * _RBLK, _RBLK)
                pltpu.make_async_copy(
                    flat_any.at[pl.ds(srow, _RBLK)], dst, sem_bulk).start()

            @pl.when(hard_ref[i] != 0)
            def _(i=i):
                def row(r, carry):
                    p = i * _RBLK + r
                    idx = p
                    for j in range(_B):
                        idx = idx + jnp.where(p >= cum_ref[j], dlt_ref[j], 0)
                    pltpu.make_async_copy(
                        flat_any.at[pl.ds(idx, 1)],
                        out_any.at[pl.ds(p, 1)], sem_row).start()
                    return carry

                lax.fori_loop(0, _RBLK, row, 0)

    for i in range(_NBLK):
        dst = out_any.at[pl.ds(i * _RBLK, _RBLK)]
        if i < _NSC:
            pltpu.make_async_copy(
                sc_any.at[pl.ds(0, _RBLK)], dst, sem_bulk).wait()
        else:
            @pl.when(hard_ref[i] == 0)
            def _(dst=dst):
                pltpu.make_async_copy(
                    flat_any.at[pl.ds(0, _RBLK)], dst, sem_bulk).wait()

            @pl.when(hard_ref[i] != 0)
            def _(dst=dst):
                pltpu.make_async_copy(
                    flat_any.at[pl.ds(0, _RBLK)], dst, sem_row).wait()


def kernel(input_tensor, seqlen):
    b, maxlen, h = input_tensor.shape
    flat = input_tensor.reshape(b * maxlen, h)
    sl = jnp.asarray(seqlen, jnp.int32)
    cum = jnp.cumsum(sl).astype(jnp.int32)
    delta = (jnp.int32(maxlen) - sl).astype(jnp.int32)

    cum_b = jnp.broadcast_to(cum[:, None], (_B, _LANES)).astype(jnp.int32)
    delta_b = jnp.broadcast_to(delta[:, None], (_B, _LANES)).astype(jnp.int32)
    mesh = plsc.VectorSubcoreMesh(core_axis_name="c", subcore_axis_name="s")
    sc_fn = pl.kernel(
        _sc_body,
        out_type=jax.ShapeDtypeStruct((_S, _H), jnp.float32),
        mesh=mesh,
        scratch_types=[
            pltpu.VMEM((2, _B, _LANES), jnp.int32),
            pltpu.VMEM((_NCHUNK, _CHUNK), jnp.int32),
            pltpu.VMEM((_NBUF, _CHUNK, _H), jnp.float32),
        ] + [pltpu.SemaphoreType.DMA] * (2 * _NBUF),
        name="unpad_sc",
    )
    sc_out = sc_fn(flat, cum_b, delta_b)

    pos0 = jnp.arange(_NBLK, dtype=jnp.int32) * _RBLK
    idx0 = pos0 + jnp.sum(
        (pos0[:, None] >= cum[None, :]) * delta[None, :], axis=1,
        dtype=jnp.int32)
    pe = pos0 + (_RBLK - 1)
    crossing = jnp.any(
        (pos0[:, None] < cum[None, :]) & (cum[None, :] <= pe[:, None]),
        axis=1)
    hard = (crossing | ((idx0 % _RBLK) != 0)).astype(jnp.int32)
    src = jnp.where(hard == 0, idx0 // _RBLK, 0).astype(jnp.int32)

    grid_spec = pltpu.PrefetchScalarGridSpec(
        num_scalar_prefetch=4,
        grid=(1,),
        in_specs=[
            pl.BlockSpec(memory_space=pl.ANY),
            pl.BlockSpec(memory_space=pl.ANY),
        ],
        out_specs=pl.BlockSpec(memory_space=pl.ANY),
        scratch_shapes=[pltpu.SemaphoreType.DMA, pltpu.SemaphoreType.DMA],
    )
    return pl.pallas_call(
        _tc_body,
        grid_spec=grid_spec,
        out_shape=jax.ShapeDtypeStruct((_TOTAL, _H), jnp.float32),
    )(src, hard, cum, delta, flat, sc_out)


# speedup vs baseline: 23.0335x; 23.0335x over previous
import jax
import jax.numpy as jnp
from jax import lax
from jax.experimental import pallas as pl
from jax.experimental.pallas import tpu as pltpu
from jax.experimental.pallas import tpu_sc as plsc

_MAXSEQLEN = 2048
_B = 8
_H = 1024
_TOTAL = _B * _MAXSEQLEN // 2
_NC = 2
_NS = 16
_NW = _NC * _NS
_ROWS_PER_W = _TOTAL // _NW
_CHUNK = 16
_NCHUNK = _ROWS_PER_W // _CHUNK
_NBUF = 6
_LANES = 16


def _unpad_body(flat_hbm, cum_hbm, delta_hbm, out_hbm,
                tbl_v, idx_v, rows_v, *sems):
    wid = lax.axis_index("s") * _NC + lax.axis_index("c")
    base = pl.multiple_of(wid * _ROWS_PER_W, _ROWS_PER_W)

    pltpu.sync_copy(cum_hbm, tbl_v.at[0])
    pltpu.sync_copy(delta_hbm, tbl_v.at[1])

    for g in range(_ROWS_PER_W // _LANES):
        pos = base + g * _LANES + lax.iota(jnp.int32, _LANES)
        acc = pos
        for j in range(_B):
            cum_j = tbl_v[0, j, :]
            dlt_j = tbl_v[1, j, :]
            acc = acc + jnp.where(pos >= cum_j, dlt_j, 0)
        gpc = _CHUNK // _LANES
        idx_v[g // gpc, pl.ds((g % gpc) * _LANES, _LANES)] = acc

    gsems = sems[:_NBUF]
    ssems = sems[_NBUF:]

    def start_gather(c, buf):
        cp = pltpu.make_async_copy(
            flat_hbm.at[idx_v.at[c]], rows_v.at[buf], gsems[buf])
        cp.start()
        return cp

    g_handles = [None] * _NCHUNK
    s_handles = [None] * _NBUF
    for c in range(min(_NBUF - 1, _NCHUNK)):
        g_handles[c] = start_gather(c, c % _NBUF)
    for c in range(_NCHUNK):
        buf = c % _NBUF
        g_handles[c].wait()
        cp = pltpu.make_async_copy(
            rows_v.at[buf],
            out_hbm.at[pl.ds(base + c * _CHUNK, _CHUNK)],
            ssems[buf])
        cp.start()
        s_handles[buf] = cp
        nxt = c + _NBUF - 1
        if nxt < _NCHUNK:
            nb = nxt % _NBUF
            if s_handles[nb] is not None:
                s_handles[nb].wait()
                s_handles[nb] = None
            g_handles[nxt] = start_gather(nxt, nb)
    for buf in range(_NBUF):
        if s_handles[buf] is not None:
            s_handles[buf].wait()


def kernel(input_tensor, seqlen):
    b, maxlen, h = input_tensor.shape
    flat = input_tensor.reshape(b * maxlen, h)
    sl = jnp.asarray(seqlen, jnp.int32)
    cum = jnp.cumsum(sl)
    delta = jnp.int32(maxlen) - sl
    cum_b = jnp.broadcast_to(cum[:, None], (_B, _LANES)).astype(jnp.int32)
    delta_b = jnp.broadcast_to(delta[:, None], (_B, _LANES)).astype(jnp.int32)

    mesh = plsc.VectorSubcoreMesh(core_axis_name="c", subcore_axis_name="s")
    fn = pl.kernel(
        _unpad_body,
        out_type=jax.ShapeDtypeStruct((_TOTAL, _H), jnp.float32),
        mesh=mesh,
        scratch_types=[
            pltpu.VMEM((2, _B, _LANES), jnp.int32),
            pltpu.VMEM((_NCHUNK, _CHUNK), jnp.int32),
            pltpu.VMEM((_NBUF, _CHUNK, _H), jnp.float32),
        ] + [pltpu.SemaphoreType.DMA] * (2 * _NBUF),
    )
    return fn(flat, cum_b, delta_b)
